# Initial kernel scaffold; baseline (speedup 1.0000x reference)
import jax
import jax.numpy as jnp
from jax.experimental import pallas as pl


def kernel(input, target, target_padding_mask):
    raise NotImplementedError("write your pallas kernel here")


# speedup vs baseline: 1.5862x; 1.5862x over previous
import numpy as np
import jax
import jax.numpy as jnp
from jax.experimental import pallas as pl
from jax.experimental.pallas import tpu as pltpu

_DEEP = 256
_TYPE = 10
_PARAM = 12
_F = _DEEP + _TYPE + _PARAM

_api_lists = [[0], [0, 1], [1, 2], [3], [4, 5], [6], [7, 8], [9], [10], [11]]
_API_NP = np.zeros((_TYPE, _PARAM), dtype=np.float32)
for _t, _lst in enumerate(_api_lists):
    for _p in _lst:
        _API_NP[_t, _p] = 1.0

_ROWS = 2048


def _body(x_ref, t_ref, api_ref, o_ref):
    i = pl.program_id(0)
    x = x_ref[...]
    t = t_ref[...]

    pad = t[:, _DEEP:_DEEP + 1] == -1.0
    validf = jnp.where(pad, 0.0, 1.0)[:, 0]

    xs = x[:, :_DEEP]
    ts = t[:, :_DEEP]
    ds = jnp.where(pad, -100.0 - ts, xs - ts)
    s_svg = jnp.sum(ds * ds)

    xt = x[:, _DEEP:_DEEP + _TYPE]
    tt = t[:, _DEEP:_DEEP + _TYPE]
    m = jnp.max(xt, axis=1, keepdims=True)
    lse = m[:, 0] + jnp.log(jnp.sum(jnp.exp(xt - m), axis=1))
    picked = jnp.sum(xt * tt, axis=1)
    s_type = jnp.sum((lse - picked) * validf)
    cnt = jnp.sum(validf)

    xp = x[:, _DEEP + _TYPE:]
    tp = t[:, _DEEP + _TYPE:]
    copy = jnp.dot(tt, api_ref[...], preferred_element_type=jnp.float32) > 0.5
    dp = jnp.where(pad, -100.0 - tp, jnp.where(copy, 0.0, xp - tp))
    s_param = jnp.sum(dp * dp)

    @pl.when(i == 0)
    def _init():
        o_ref[0] = 0.0
        o_ref[1] = 0.0
        o_ref[2] = 0.0
        o_ref[3] = 0.0

    o_ref[0] += s_svg
    o_ref[1] += s_type
    o_ref[2] += cnt
    o_ref[3] += s_param


def kernel(input, target, target_padding_mask):
    n = input.shape[0] * input.shape[1]
    x = input.reshape(n, _F)
    t = target.reshape(n, _F)
    sums = pl.pallas_call(
        _body,
        grid=(n // _ROWS,),
        in_specs=[
            pl.BlockSpec((_ROWS, _F), lambda i: (i, 0)),
            pl.BlockSpec((_ROWS, _F), lambda i: (i, 0)),
            pl.BlockSpec((_TYPE, _PARAM), lambda i: (0, 0)),
        ],
        out_specs=pl.BlockSpec(memory_space=pltpu.SMEM),
        out_shape=jax.ShapeDtypeStruct((4,), jnp.float32),
    )(x, t, jnp.asarray(_API_NP))
    loss = (10.0 * sums[0] / (n * _DEEP)
            + 0.1 * sums[1] / jnp.maximum(sums[2], 1.0)
            + sums[3] / (n * _PARAM))
    return loss
